# baseline (speedup 1.0000x reference)
import jax
import jax.numpy as jnp
from jax.experimental import pallas as pl
from jax.experimental.pallas import tpu as pltpu

D = 12
N = 2 ** D - 1
B = 32
IN_DIM = 128
H = 128


def _sigmoid(v):
    return 0.5 * jnp.tanh(0.5 * v) + 0.5


def _dot(a, b):
    return jax.lax.dot_general(
        a, b, dimension_numbers=(((1,), (1,)), ((), ())),
        preferred_element_type=jnp.float32)


G = 4


def _tree_gru_kernel(x_hbm, wxc_ref, bxc_ref, wglr_ref, wclr_ref,
                     out_ref, outt_ref, xbufs, sems):
    i = pl.program_id(0)
    nprog = B // G

    def x_copy(g, slot, pidx):
        return pltpu.make_async_copy(
            x_hbm.at[:, pidx * G + g, :], xbufs[g].at[slot], sems.at[g, slot])

    @pl.when(i == 0)
    def _():
        for g in range(G):
            x_copy(g, 0, 0).start()

    @pl.when(i + 1 < nprog)
    def _():
        for g in range(G):
            x_copy(g, (i + 1) % 2, i + 1).start()

    slot = i % 2
    for g in range(G):
        x_copy(g, slot, i).wait()
    x_refs = [xbufs[g].at[slot] for g in range(G)]

    wxc = wxc_ref[...]
    bxc = bxc_ref[...]
    wglr = wglr_ref[...]
    wclr = wclr_ref[...]

    n = 2 ** (D - 1)
    start = n - 1
    hid = [None] * G
    for g in range(G):
        x = x_refs[g][start:start + n, :]
        a = _dot(x, wxc[4 * H:, :]) + bxc[:, 4 * H:]
        hid[g] = _sigmoid(a[:, :H]) * jnp.tanh(a[:, H:])
        out_ref[g, start - 1:start - 1 + n, :] = hid[g]

    for level in range(D - 2, -1, -1):
        n = 2 ** level
        start = n - 1
        for g in range(G):
            x = x_refs[g][start:start + n, :]
            ch2 = hid[g].reshape(n, 2 * H)
            a = _dot(x, wxc) + bxc
            gates = _sigmoid(a[:, :5 * H] + _dot(ch2, wglr))
            gated2 = gates[:, :2 * H] * ch2
            cell = jnp.tanh(a[:, 5 * H:] + _dot(gated2, wclr))
            z2 = gates[:, 2 * H:4 * H] * ch2
            hid[g] = z2[:, :H] + z2[:, H:] + gates[:, 4 * H:] * cell
            if level > 0:
                out_ref[g, start - 1:start - 1 + n, :] = hid[g]
            else:
                out_ref[g, N - 1:N, :] = hid[g]
                outt_ref[g, :, :] = hid[g]


def kernel(inputs, W_gih, b_gih, W_glhh, W_grhh, W_cih, b_cih, W_clhh, W_crhh):
    wxc = jnp.concatenate([W_gih, W_cih], axis=0)
    bxc = jnp.concatenate([b_gih, b_cih]).reshape(1, 6 * H)
    wglr = jnp.concatenate([W_glhh, W_grhh], axis=1)
    wclr = jnp.concatenate([W_clhh, W_crhh], axis=1)

    outputs, output_t = pl.pallas_call(
        _tree_gru_kernel,
        grid=(B // G,),
        in_specs=[
            pl.BlockSpec(memory_space=pl.ANY),
            pl.BlockSpec((6 * H, IN_DIM), lambda b: (0, 0)),
            pl.BlockSpec((1, 6 * H), lambda b: (0, 0)),
            pl.BlockSpec((5 * H, 2 * H), lambda b: (0, 0)),
            pl.BlockSpec((H, 2 * H), lambda b: (0, 0)),
        ],
        out_specs=[
            pl.BlockSpec((G, N, H), lambda b: (b, 0, 0)),
            pl.BlockSpec((G, 1, H), lambda b: (b, 0, 0)),
        ],
        out_shape=[
            jax.ShapeDtypeStruct((B, N, H), jnp.float32),
            jax.ShapeDtypeStruct((B, 1, H), jnp.float32),
        ],
        scratch_shapes=[
            [pltpu.VMEM((2, N, IN_DIM), jnp.float32) for _ in range(G)],
            pltpu.SemaphoreType.DMA((G, 2)),
        ],
    )(inputs, wxc, bxc, wglr, wclr)
    return outputs, output_t.reshape(B, H)


# speedup vs baseline: 1.0321x; 1.0321x over previous
import jax
import jax.numpy as jnp
from jax.experimental import pallas as pl
from jax.experimental.pallas import tpu as pltpu

D = 12
N = 2 ** D - 1
B = 32
IN_DIM = 128
H = 128


def _sigmoid(v):
    return 0.5 * jnp.tanh(0.5 * v) + 0.5


def _dot(a, b):
    return jax.lax.dot_general(
        a, b, dimension_numbers=(((1,), (1,)), ((), ())),
        preferred_element_type=jnp.float32)


G = 2


def _tree_gru_kernel(x_hbm, wxc_ref, bxc_ref, wglr_ref, wclr_ref,
                     out_ref, outt_ref, xbufs, sems):
    i = pl.program_id(0)
    nprog = B // G

    def x_copy(g, slot, pidx):
        return pltpu.make_async_copy(
            x_hbm.at[:, pidx * G + g, :], xbufs[g].at[slot], sems.at[g, slot])

    @pl.when(i == 0)
    def _():
        for g in range(G):
            x_copy(g, 0, 0).start()

    @pl.when(i + 1 < nprog)
    def _():
        for g in range(G):
            x_copy(g, (i + 1) % 2, i + 1).start()

    slot = i % 2
    for g in range(G):
        x_copy(g, slot, i).wait()
    x_refs = [xbufs[g].at[slot] for g in range(G)]

    wxc = wxc_ref[...]
    bxc = bxc_ref[...]
    wglr = wglr_ref[...]
    wclr = wclr_ref[...]

    n = 2 ** (D - 1)
    start = n - 1
    hid = [None] * G
    for g in range(G):
        x = x_refs[g][start:start + n, :]
        a = _dot(x, wxc[4 * H:, :]) + bxc[:, 4 * H:]
        hid[g] = 0.5 * (jnp.tanh(a[:, :H]) + 1.0) * jnp.tanh(a[:, H:])
        out_ref[g, start - 1:start - 1 + n, :] = hid[g]

    for level in range(D - 2, -1, -1):
        n = 2 ** level
        start = n - 1
        for g in range(G):
            x = x_refs[g][start:start + n, :]
            ch2 = hid[g].reshape(n, 2 * H)
            a = _dot(x, wxc) + bxc
            t = jnp.tanh(a[:, :5 * H] + _dot(ch2, wglr))
            p2 = (t[:, :2 * H] + 1.0) * ch2
            cell = jnp.tanh(a[:, 5 * H:] + _dot(p2, wclr))
            q2 = (t[:, 2 * H:4 * H] + 1.0) * ch2
            hid[g] = 0.5 * (q2[:, :H] + q2[:, H:]
                            + (t[:, 4 * H:] + 1.0) * cell)
            if level > 0:
                out_ref[g, start - 1:start - 1 + n, :] = hid[g]
            else:
                out_ref[g, N - 1:N, :] = hid[g]
                outt_ref[g, :, :] = hid[g]


def kernel(inputs, W_gih, b_gih, W_glhh, W_grhh, W_cih, b_cih, W_clhh, W_crhh):
    wxc = jnp.concatenate([0.5 * W_gih, W_cih], axis=0)
    bxc = jnp.concatenate([0.5 * b_gih, b_cih]).reshape(1, 6 * H)
    wglr = 0.5 * jnp.concatenate([W_glhh, W_grhh], axis=1)
    wclr = 0.5 * jnp.concatenate([W_clhh, W_crhh], axis=1)

    outputs, output_t = pl.pallas_call(
        _tree_gru_kernel,
        grid=(B // G,),
        in_specs=[
            pl.BlockSpec(memory_space=pl.ANY),
            pl.BlockSpec((6 * H, IN_DIM), lambda b: (0, 0)),
            pl.BlockSpec((1, 6 * H), lambda b: (0, 0)),
            pl.BlockSpec((5 * H, 2 * H), lambda b: (0, 0)),
            pl.BlockSpec((H, 2 * H), lambda b: (0, 0)),
        ],
        out_specs=[
            pl.BlockSpec((G, N, H), lambda b: (b, 0, 0)),
            pl.BlockSpec((G, 1, H), lambda b: (b, 0, 0)),
        ],
        out_shape=[
            jax.ShapeDtypeStruct((B, N, H), jnp.float32),
            jax.ShapeDtypeStruct((B, 1, H), jnp.float32),
        ],
        scratch_shapes=[
            [pltpu.VMEM((2, N, IN_DIM), jnp.float32) for _ in range(G)],
            pltpu.SemaphoreType.DMA((G, 2)),
        ],
    )(inputs, wxc, bxc, wglr, wclr)
    return outputs, output_t.reshape(B, H)
